# baseline (speedup 1.0000x reference)
import functools

import jax
import jax.numpy as jnp
from jax import lax
from jax.experimental import pallas as pl
from jax.experimental.pallas import tpu as pltpu
from jax.experimental.pallas import tpu_sc as plsc

_B = 128
_V = 100000
_NC = 2
_NS = 16
_NW = _NC * _NS
_RPW = _B // _NW
_CHUNK = 20000
_NCH = _V // _CHUNK
_L = 16
_NVREG = _CHUNK // _L

_mesh = plsc.VectorSubcoreMesh(core_axis_name="c", subcore_axis_name="s")


@functools.partial(
    pl.kernel,
    out_type=jax.ShapeDtypeStruct((_NW * _L,), jnp.float32),
    mesh=_mesh,
    compiler_params=pltpu.CompilerParams(needs_layout_passes=False),
    scratch_types=[
        pltpu.VMEM((_CHUNK,), jnp.float32),
        pltpu.VMEM((_CHUNK,), jnp.float32),
        pltpu.VMEM((_RPW, _L), jnp.int32),
        pltpu.VMEM((_RPW, _L), jnp.float32),
        pltpu.VMEM((_L,), jnp.float32),
        pltpu.SemaphoreType.DMA,
        pltpu.SemaphoreType.DMA,
        pltpu.SemaphoreType.DMA,
    ],
)
def _row_flags(pred_hbm, tgt_hbm, out_hbm, buf0, buf1, t16_v, tval_v, out_v,
               sem0, sem1, semg):
    wid = lax.axis_index("s") * _NC + lax.axis_index("c")
    row0 = wid * _RPW

    tgt_gathers = [
        pltpu.async_copy(
            tgt_hbm.at[jnp.full((_L,), row0 + r, jnp.int32)],
            t16_v.at[r], semg)
        for r in range(_RPW)
    ]
    for g in tgt_gathers:
        g.wait()
    tvecs = []
    gathers = []
    for r in range(_RPW):
        tvec = t16_v[r]
        gidx = jnp.full((_L,), (row0 + r) * _V, jnp.int32) + tvec
        gathers.append(
            pltpu.async_copy(pred_hbm.at[gidx], tval_v.at[r], semg))
        tvecs.append(tvec)

    bufs = (buf0, buf1)
    sems = (sem0, sem1)
    segs = [(r, c) for r in range(_RPW) for c in range(_NCH)]

    def seg_slice(k):
        r, c = segs[k]
        start = pl.multiple_of((row0 + r) * _V + c * _CHUNK, 8)
        return pred_hbm.at[pl.ds(start, _CHUNK)]

    def issue(k):
        pltpu.async_copy(seg_slice(k), bufs[k % 2], sems[k % 2])

    issue(0)
    acc = jnp.zeros((_L,), jnp.float32)
    best = biter = None
    for k, (r, c) in enumerate(segs):
        if k + 1 < len(segs):
            issue(k + 1)
        pltpu.make_async_copy(seg_slice(k), bufs[k % 2], sems[k % 2]).wait()

        if c == 0:
            best = jnp.full((_L,), -jnp.inf, jnp.float32)
            biter = jnp.zeros((_L,), jnp.int32)
        buf = bufs[k % 2]
        base = c * _NVREG

        def body(i, st, buf=buf, base=base):
            bb, bi = st
            v = buf[pl.ds(i * _L, _L)]
            m = v > bb
            bb = jnp.where(m, v, bb)
            bi = jnp.where(m, jnp.full((_L,), base + i, jnp.int32), bi)
            return bb, bi

        best, biter = plsc.parallel_loop(
            0, _NVREG, unroll=8, carry=(best, biter))(body)

        if c == _NCH - 1:
            if r == 0:
                for g in gathers:
                    g.wait()
            idx16 = biter * _L + lax.iota(jnp.int32, _L)
            tval = tval_v[r]
            tv = tvecs[r]
            beats = (best > tval) | ((best == tval) & (idx16 < tv))
            nbeat = plsc.all_reduce_population_count(beats)
            flag = (nbeat == 0) & (lax.iota(jnp.int32, _L) == r)
            acc = acc + jnp.where(flag, 1.0, 0.0).astype(jnp.float32)

    out_v[...] = acc
    pltpu.sync_copy(out_v, out_hbm.at[pl.ds(wid * _L, _L)])


@functools.partial(
    pl.kernel,
    out_type=jax.ShapeDtypeStruct((_L,), jnp.float32),
    mesh=_mesh,
    compiler_params=pltpu.CompilerParams(needs_layout_passes=False),
    scratch_types=[
        pltpu.VMEM((_NW * _L,), jnp.float32),
        pltpu.VMEM((_L,), jnp.float32),
    ],
)
def _finalize(flags_hbm, out_hbm, fv, ov):
    wid = lax.axis_index("s") * _NC + lax.axis_index("c")

    @pl.when(wid == 0)
    def _():
        pltpu.sync_copy(flags_hbm, fv)
        tot = jnp.zeros((_L,), jnp.int32)
        for i in range(_NW):
            bv = fv[pl.ds(i * _L, _L)] != 0.0
            tot = tot + plsc.all_reduce_population_count(bv)
        s = tot.astype(jnp.float32) * (100.0 / _B)
        ov[...] = s
        pltpu.sync_copy(ov, out_hbm)


def kernel(pred, target):
    flags = _row_flags(pred.reshape(-1), target)
    res = _finalize(flags)
    return (res[:1],)


# speedup vs baseline: 1.6886x; 1.6886x over previous
import functools

import jax
import jax.numpy as jnp
from jax import lax
from jax.experimental import pallas as pl
from jax.experimental.pallas import tpu as pltpu
from jax.experimental.pallas import tpu_sc as plsc

_B = 128
_V = 100000
_NC = 2
_NS = 16
_NW = _NC * _NS
_L = 16
_TR = 8
_HALF_T = 391
_HALF_C = _HALF_T * 128
_CHT = 23
_CHC = _CHT * 128
_NCH = _HALF_T // _CHT
_NVREG = _CHC // _L
_IMAX = 2**31 - 1

_mesh = plsc.VectorSubcoreMesh(core_axis_name="c", subcore_axis_name="s")


@functools.partial(
    pl.kernel,
    out_type=(
        jax.ShapeDtypeStruct((_NW * _L,), jnp.float32),
        jax.ShapeDtypeStruct((_NW * _L,), jnp.int32),
    ),
    mesh=_mesh,
    compiler_params=pltpu.CompilerParams(needs_layout_passes=False),
    scratch_types=[
        pltpu.VMEM((_TR, _CHC), jnp.float32),
        pltpu.VMEM((_TR, _CHC), jnp.float32),
        pltpu.VMEM((_L,), jnp.float32),
        pltpu.VMEM((_L,), jnp.int32),
        pltpu.SemaphoreType.DMA,
        pltpu.SemaphoreType.DMA,
    ],
)
def _partial_argmax(pred_hbm, max_hbm, idx_hbm, buf0, buf1, vm_v, vi_v,
                    sem0, sem1):
    wid = lax.axis_index("s") * _NC + lax.axis_index("c")
    tr = wid // 2
    half = wid % 2
    row_base = pl.multiple_of(tr * _TR, _TR)
    col0 = half * _HALF_C

    bufs = (buf0, buf1)
    sems = (sem0, sem1)

    def seg_slice(c):
        start = pl.multiple_of(col0 + c * _CHC, 128)
        return pred_hbm.at[pl.ds(row_base, _TR), pl.ds(start, _CHC)]

    def issue(c):
        pltpu.async_copy(seg_slice(c), bufs[c % 2], sems[c % 2])

    issue(0)
    best = [jnp.full((_L,), -jnp.inf, jnp.float32) for _ in range(_TR)]
    bcol = [jnp.zeros((_L,), jnp.int32) for _ in range(_TR)]
    for c in range(_NCH):
        if c + 1 < _NCH:
            issue(c + 1)
        pltpu.make_async_copy(seg_slice(c), bufs[c % 2], sems[c % 2]).wait()
        buf = bufs[c % 2]
        cbase = col0 + c * _CHC
        masked = c == _NCH - 1

        for r in range(_TR):
            def body(i, st, buf=buf, r=r, cbase=cbase, masked=masked):
                bb, bc = st
                v = buf[r, pl.ds(i * _L, _L)]
                cb = cbase + i * _L
                if masked:
                    valid = jnp.full((_L,), cb, jnp.int32) \
                        + lax.iota(jnp.int32, _L) < _V
                    v = jnp.where(valid, v, -jnp.inf)
                m = v > bb
                bb = jnp.where(m, v, bb)
                bc = jnp.where(m, jnp.full((_L,), cb, jnp.int32), bc)
                return bb, bc

            best[r], bcol[r] = plsc.parallel_loop(
                0, _NVREG, unroll=8, carry=(best[r], bcol[r]))(body)

    vm = jnp.full((_L,), -jnp.inf, jnp.float32)
    vi = jnp.full((_L,), _IMAX, jnp.int32)
    lane = lax.iota(jnp.int32, _L)
    for r in range(_TR):
        rmax = jnp.max(best[r])
        idx16 = bcol[r] + lane
        argidx = jnp.min(jnp.where(best[r] == rmax, idx16, _IMAX))
        dst = lane == (tr % 2) * _TR + r
        vm = jnp.where(dst, rmax, vm)
        vi = jnp.where(dst, argidx, vi)

    vm_v[...] = vm
    vi_v[...] = vi
    pltpu.sync_copy(vm_v, max_hbm.at[pl.ds(wid * _L, _L)])
    pltpu.sync_copy(vi_v, idx_hbm.at[pl.ds(wid * _L, _L)])


@functools.partial(
    pl.kernel,
    out_type=jax.ShapeDtypeStruct((_L,), jnp.float32),
    mesh=_mesh,
    compiler_params=pltpu.CompilerParams(needs_layout_passes=False),
    scratch_types=[
        pltpu.VMEM((_NW * _L,), jnp.float32),
        pltpu.VMEM((_NW * _L,), jnp.int32),
        pltpu.VMEM((_B,), jnp.int32),
        pltpu.VMEM((_L,), jnp.float32),
    ],
)
def _finalize(max_hbm, idx_hbm, tgt_hbm, out_hbm, mv_v, iv_v, tv_v, ov):
    wid = lax.axis_index("s") * _NC + lax.axis_index("c")

    @pl.when(wid == 0)
    def _():
        pltpu.sync_copy(max_hbm, mv_v)
        pltpu.sync_copy(idx_hbm, iv_v)
        pltpu.sync_copy(tgt_hbm, tv_v)

        def comb(a, b):
            (m0, i0), (m1, i1) = a, b
            pick = (m0 > m1) | ((m0 == m1) & (i0 < i1))
            return jnp.where(pick, m0, m1), jnp.where(pick, i0, i1)

        tot = jnp.zeros((_L,), jnp.int32)
        for g in range(_B // _L):
            parts = []
            for j in range(4):
                o = (4 * g + j) * _L
                parts.append((mv_v[pl.ds(o, _L)], iv_v[pl.ds(o, _L)]))
            _, widx = comb(comb(parts[0], parts[1]),
                           comb(parts[2], parts[3]))
            tv = tv_v[pl.ds(g * _L, _L)]
            tot = tot + plsc.all_reduce_population_count(widx == tv)
        ov[...] = tot.astype(jnp.float32) * (100.0 / _B)
        pltpu.sync_copy(ov, out_hbm)


def kernel(pred, target):
    maxes, idxs = _partial_argmax(pred)
    res = _finalize(maxes, idxs, target)
    return (res[:1],)


# speedup vs baseline: 1.6906x; 1.0012x over previous
import functools

import jax
import jax.numpy as jnp
from jax import lax
from jax.experimental import pallas as pl
from jax.experimental.pallas import tpu as pltpu
from jax.experimental.pallas import tpu_sc as plsc

_B = 128
_V = 100000
_NC = 2
_NS = 16
_NW = _NC * _NS
_L = 16
_TR = 8
_HALF_T = 391
_HALF_C = _HALF_T * 128
_CHT = 23
_CHC = _CHT * 128
_NCH = _HALF_T // _CHT
_NVREG = _CHC // _L
_IMAX = 2**31 - 1

_mesh = plsc.VectorSubcoreMesh(core_axis_name="c", subcore_axis_name="s")


@functools.partial(
    pl.kernel,
    out_type=(
        jax.ShapeDtypeStruct((_NW * _L,), jnp.float32),
        jax.ShapeDtypeStruct((_NW * _L,), jnp.int32),
    ),
    mesh=_mesh,
    compiler_params=pltpu.CompilerParams(
        needs_layout_passes=False, use_tc_tiling_on_sc=True),
    scratch_types=[
        pltpu.VMEM((_TR, _CHC), jnp.float32),
        pltpu.VMEM((_TR, _CHC), jnp.float32),
        pltpu.VMEM((_L,), jnp.float32),
        pltpu.VMEM((_L,), jnp.int32),
        pltpu.SemaphoreType.DMA,
        pltpu.SemaphoreType.DMA,
    ],
)
def _partial_argmax(pred_hbm, max_hbm, idx_hbm, buf0, buf1, vm_v, vi_v,
                    sem0, sem1):
    wid = lax.axis_index("s") * _NC + lax.axis_index("c")
    tr = wid // 2
    half = wid % 2
    row_base = pl.multiple_of(tr * _TR, _TR)
    col0 = half * _HALF_C

    bufs = (buf0, buf1)
    sems = (sem0, sem1)

    def seg_slice(c):
        start = pl.multiple_of(col0 + c * _CHC, 128)
        return pred_hbm.at[pl.ds(row_base, _TR), pl.ds(start, _CHC)]

    def issue(c):
        pltpu.async_copy(seg_slice(c), bufs[c % 2], sems[c % 2])

    issue(0)
    best = [jnp.full((_L,), -jnp.inf, jnp.float32) for _ in range(_TR)]
    bcol = [jnp.zeros((_L,), jnp.int32) for _ in range(_TR)]
    for c in range(_NCH):
        if c + 1 < _NCH:
            issue(c + 1)
        pltpu.make_async_copy(seg_slice(c), bufs[c % 2], sems[c % 2]).wait()
        buf = bufs[c % 2]
        cbase = col0 + c * _CHC
        masked = c == _NCH - 1

        for r in range(_TR):
            def body(i, st, buf=buf, r=r, cbase=cbase, masked=masked):
                bb, bc = st
                v = buf[r, pl.ds(i * _L, _L)]
                cb = cbase + i * _L
                if masked:
                    valid = jnp.full((_L,), cb, jnp.int32) \
                        + lax.iota(jnp.int32, _L) < _V
                    v = jnp.where(valid, v, -jnp.inf)
                m = v > bb
                bb = jnp.where(m, v, bb)
                bc = jnp.where(m, jnp.full((_L,), cb, jnp.int32), bc)
                return bb, bc

            best[r], bcol[r] = plsc.parallel_loop(
                0, _NVREG, unroll=8, carry=(best[r], bcol[r]))(body)

    vm = jnp.full((_L,), -jnp.inf, jnp.float32)
    vi = jnp.full((_L,), _IMAX, jnp.int32)
    lane = lax.iota(jnp.int32, _L)
    for r in range(_TR):
        rmax = jnp.max(best[r])
        idx16 = bcol[r] + lane
        argidx = jnp.min(jnp.where(best[r] == rmax, idx16, _IMAX))
        dst = lane == (tr % 2) * _TR + r
        vm = jnp.where(dst, rmax, vm)
        vi = jnp.where(dst, argidx, vi)

    vm_v[...] = vm
    vi_v[...] = vi
    pltpu.sync_copy(vm_v, max_hbm.at[pl.ds(wid * _L, _L)])
    pltpu.sync_copy(vi_v, idx_hbm.at[pl.ds(wid * _L, _L)])


@functools.partial(
    pl.kernel,
    out_type=jax.ShapeDtypeStruct((_L,), jnp.float32),
    mesh=_mesh,
    compiler_params=pltpu.CompilerParams(needs_layout_passes=False),
    scratch_types=[
        pltpu.VMEM((_NW * _L,), jnp.float32),
        pltpu.VMEM((_NW * _L,), jnp.int32),
        pltpu.VMEM((_B,), jnp.int32),
        pltpu.VMEM((_L,), jnp.float32),
    ],
)
def _finalize(max_hbm, idx_hbm, tgt_hbm, out_hbm, mv_v, iv_v, tv_v, ov):
    wid = lax.axis_index("s") * _NC + lax.axis_index("c")

    @pl.when(wid == 0)
    def _():
        pltpu.sync_copy(max_hbm, mv_v)
        pltpu.sync_copy(idx_hbm, iv_v)
        pltpu.sync_copy(tgt_hbm, tv_v)

        def comb(a, b):
            (m0, i0), (m1, i1) = a, b
            pick = (m0 > m1) | ((m0 == m1) & (i0 < i1))
            return jnp.where(pick, m0, m1), jnp.where(pick, i0, i1)

        tot = jnp.zeros((_L,), jnp.int32)
        for g in range(_B // _L):
            parts = []
            for j in range(4):
                o = (4 * g + j) * _L
                parts.append((mv_v[pl.ds(o, _L)], iv_v[pl.ds(o, _L)]))
            _, widx = comb(comb(parts[0], parts[1]),
                           comb(parts[2], parts[3]))
            tv = tv_v[pl.ds(g * _L, _L)]
            tot = tot + plsc.all_reduce_population_count(widx == tv)
        ov[...] = tot.astype(jnp.float32) * (100.0 / _B)
        pltpu.sync_copy(ov, out_hbm)


def kernel(pred, target):
    maxes, idxs = _partial_argmax(pred)
    res = _finalize(maxes, idxs, target)
    return (res[:1],)


# speedup vs baseline: 3.7583x; 2.2231x over previous
import functools

import jax
import jax.numpy as jnp
from jax import lax
from jax.experimental import pallas as pl
from jax.experimental.pallas import tpu as pltpu
from jax.experimental.pallas import tpu_sc as plsc

_B = 128
_V = 100000
_NC = 2
_NS = 16
_NW = _NC * _NS
_L = 16
_NG = _B // _L
_NT = _V // 8
_WT = 391
_LAST0 = _NT - _WT
_CHT = 23
_CHCLS = _CHT * 8
_NCH = _WT // _CHT
_IMAX = 2**31 - 1

_mesh = plsc.VectorSubcoreMesh(core_axis_name="c", subcore_axis_name="s")


@functools.partial(
    pl.kernel,
    out_type=(
        jax.ShapeDtypeStruct((_NW, _B), jnp.float32),
        jax.ShapeDtypeStruct((_NW, _B), jnp.int32),
    ),
    mesh=_mesh,
    compiler_params=pltpu.CompilerParams(needs_layout_passes=False),
    scratch_types=[
        pltpu.VMEM((_CHCLS, _B), jnp.float32),
        pltpu.VMEM((_CHCLS, _B), jnp.float32),
        pltpu.VMEM((_B,), jnp.float32),
        pltpu.VMEM((_B,), jnp.int32),
        pltpu.SemaphoreType.DMA,
        pltpu.SemaphoreType.DMA,
    ],
)
def _partial_argmax(predt_hbm, max_hbm, idx_hbm, buf0, buf1, vm_v, vi_v,
                    sem0, sem1):
    wid = lax.axis_index("s") * _NC + lax.axis_index("c")
    start_tile = jnp.where(wid == _NW - 1, _LAST0, wid * _WT)
    cls0 = pl.multiple_of(start_tile * 8, 8)

    bufs = (buf0, buf1)
    sems = (sem0, sem1)

    def seg_slice(c):
        start = pl.multiple_of(cls0 + c * _CHCLS, 8)
        return predt_hbm.at[pl.ds(start, _CHCLS), :]

    def issue(c):
        pltpu.async_copy(seg_slice(c), bufs[c % 2], sems[c % 2])

    issue(0)
    best = [jnp.full((_L,), -jnp.inf, jnp.float32) for _ in range(_NG)]
    bcls = [jnp.zeros((_L,), jnp.int32) for _ in range(_NG)]
    for c in range(_NCH):
        if c + 1 < _NCH:
            issue(c + 1)
        pltpu.make_async_copy(seg_slice(c), bufs[c % 2], sems[c % 2]).wait()
        buf = bufs[c % 2]
        ccls0 = cls0 + c * _CHCLS

        def body(i, st, buf=buf, ccls0=ccls0):
            bb, bc = st
            clsv = jnp.full((_L,), ccls0 + i, jnp.int32)
            nb, nc2 = [], []
            for g in range(_NG):
                v = buf[i, pl.ds(g * _L, _L)]
                m = v > bb[g]
                nb.append(jnp.where(m, v, bb[g]))
                nc2.append(jnp.where(m, clsv, bc[g]))
            return nb, nc2

        best, bcls = plsc.parallel_loop(
            0, _CHCLS, unroll=2, carry=(best, bcls))(body)

    for g in range(_NG):
        vm_v[pl.ds(g * _L, _L)] = best[g]
        vi_v[pl.ds(g * _L, _L)] = bcls[g]
    pltpu.sync_copy(vm_v, max_hbm.at[wid])
    pltpu.sync_copy(vi_v, idx_hbm.at[wid])


def _merge_body(max_ref, idx_ref, tgt_ref, out_ref):
    m = max_ref[...]
    i = idx_ref[...]
    t = tgt_ref[...]
    rm = jnp.max(m, axis=0)
    wi = jnp.min(jnp.where(m == rm[None, :], i, _IMAX), axis=0)
    cnt = jnp.sum(jnp.where(wi == t, 1.0, 0.0).astype(jnp.float32))
    out_ref[0] = cnt * (100.0 / _B)


_merge = pl.pallas_call(
    _merge_body,
    out_shape=jax.ShapeDtypeStruct((1,), jnp.float32),
    out_specs=pl.BlockSpec(memory_space=pltpu.SMEM),
)


def kernel(pred, target):
    maxes, idxs = _partial_argmax(pred.T)
    res = _merge(maxes, idxs, target)
    return (res,)
